# Initial kernel scaffold; baseline (speedup 1.0000x reference)
import jax
import jax.numpy as jnp
from jax.experimental import pallas as pl


def kernel(atom_type, r_feat, p_feat, pos, bond_index, bond_type, batch, time_step, atom_emb, atom_feat_W, bond_emb, edge_mlp_W1, edge_mlp_b1, edge_mlp_W2, edge_mlp_b2, edge_cat_W1, edge_cat_b1, edge_cat_W2, edge_cat_b2, filt_W1, filt_b1, filt_W2, filt_b2, lin_W, out_W, out_b, g_W1, g_b1, g_W2, g_b2, g_W3, g_b3):
    raise NotImplementedError("write your pallas kernel here")


# speedup vs baseline: 2.7037x; 2.7037x over previous
import functools

import jax
import jax.numpy as jnp
from jax import lax
from jax.experimental import pallas as pl
from jax.experimental.pallas import tpu as pltpu
from jax.experimental.pallas import tpu_sc as plsc

N = 10000
E = 320000
H = 64
HALF = 32
FEAT = 128
L = 3

NC = 2
NS = 16
NW = NC * NS
EC = E // NW
BE = 80
NB = EC // BE
NR = N // NS
RB = 125

_mesh = plsc.VectorSubcoreMesh(core_axis_name="c", subcore_axis_name="s")


PP = 16


@functools.partial(
    pl.kernel,
    mesh=_mesh,
    compiler_params=pltpu.CompilerParams(use_tc_tiling_on_sc=False),
    out_type=jax.ShapeDtypeStruct((E, PP), jnp.float32),
    scratch_types=[
        pltpu.VMEM((BE,), jnp.int32),
        pltpu.VMEM((BE,), jnp.int32),
        pltpu.VMEM((BE, PP), jnp.float32),
        pltpu.VMEM((BE, PP), jnp.float32),
        pltpu.SemaphoreType.DMA,
    ],
)
def _sc_geom(posp_hbm, row_hbm, col_hbm, out_hbm, rowb, colb, prb, pcb, sem):
    c = lax.axis_index("c")
    s = lax.axis_index("s")
    w = c * NS + s

    def blk(b, _):
        base = w * EC + b * BE
        pltpu.sync_copy(row_hbm.at[pl.ds(base, BE)], rowb)
        pltpu.sync_copy(col_hbm.at[pl.ds(base, BE)], colb)
        pltpu.async_copy(posp_hbm.at[rowb], prb, sem)
        pltpu.async_copy(posp_hbm.at[colb], pcb, sem)
        pltpu.make_async_copy(posp_hbm.at[rowb], prb, sem).wait()
        pltpu.make_async_copy(posp_hbm.at[colb], pcb, sem).wait()

        def body(i, _):
            d = prb[i] - pcb[i]
            prb[i] = d * d
            return 0

        lax.fori_loop(0, BE, body, 0)
        pltpu.sync_copy(prb, out_hbm.at[pl.ds(base, BE)])
        return 0

    lax.fori_loop(0, NB, blk, 0)


@functools.partial(
    pl.kernel,
    mesh=_mesh,
    compiler_params=pltpu.CompilerParams(use_tc_tiling_on_sc=False),
    out_type=jax.ShapeDtypeStruct((2 * N, H), jnp.float32),
    scratch_types=[
        pltpu.VMEM_SHARED((N, H), jnp.float32),
        pltpu.VMEM((RB, H), jnp.float32),
        pltpu.VMEM((BE,), jnp.int32),
        pltpu.VMEM((BE,), jnp.int32),
        pltpu.VMEM((BE, H), jnp.float32),
        pltpu.VMEM((BE, H), jnp.float32),
        pltpu.SemaphoreType.DMA,
    ],
)
def _sc_msg(hl_hbm, wf_hbm, row_hbm, col_hbm, out_hbm,
            acc, zob, rowb, colb, hb, wfb, sem):
    c = lax.axis_index("c")
    s = lax.axis_index("s")
    w = c * NS + s

    def zero(i, _):
        for j in range(H // 16):
            zob[i, pl.ds(j * 16, 16)] = jnp.zeros((16,), jnp.float32)
        return 0

    lax.fori_loop(0, RB, zero, 0)
    for jj in range(NR // RB):
        pltpu.sync_copy(zob, acc.at[pl.ds(s * NR + jj * RB, RB)])
    plsc.subcore_barrier()

    def blk(b, _):
        base = w * EC + b * BE
        pltpu.sync_copy(row_hbm.at[pl.ds(base, BE)], rowb)
        pltpu.sync_copy(col_hbm.at[pl.ds(base, BE)], colb)
        pltpu.async_copy(hl_hbm.at[rowb], hb, sem).wait()
        pltpu.sync_copy(wf_hbm.at[pl.ds(base, BE)], wfb)

        def mul(i, _):
            for j in range(H // 16):
                hb[i, pl.ds(j * 16, 16)] = (
                    hb[i, pl.ds(j * 16, 16)] * wfb[i, pl.ds(j * 16, 16)])
            return 0

        lax.fori_loop(0, BE, mul, 0)
        pltpu.sync_copy(hb, acc.at[colb], add=True)
        return 0

    lax.fori_loop(0, NB, blk, 0)
    plsc.subcore_barrier()
    for jj in range(NR // RB):
        off = s * NR + jj * RB
        pltpu.sync_copy(acc.at[pl.ds(off, RB)], zob)
        pltpu.sync_copy(zob, out_hbm.at[pl.ds(c * N + off, RB)])


@functools.partial(
    pl.kernel,
    mesh=_mesh,
    compiler_params=pltpu.CompilerParams(use_tc_tiling_on_sc=False),
    out_type=jax.ShapeDtypeStruct((E, H), jnp.float32),
    scratch_types=[
        pltpu.VMEM((BE,), jnp.int32),
        pltpu.VMEM((BE,), jnp.int32),
        pltpu.VMEM((BE, H), jnp.float32),
        pltpu.VMEM((BE, H), jnp.float32),
        pltpu.SemaphoreType.DMA,
    ],
)
def _sc_pair(h_hbm, row_hbm, col_hbm, out_hbm, rowb, colb, hb, cb, sem):
    c = lax.axis_index("c")
    s = lax.axis_index("s")
    w = c * NS + s

    def blk(b, _):
        base = w * EC + b * BE
        pltpu.sync_copy(row_hbm.at[pl.ds(base, BE)], rowb)
        pltpu.sync_copy(col_hbm.at[pl.ds(base, BE)], colb)
        pltpu.async_copy(h_hbm.at[rowb], hb, sem)
        pltpu.async_copy(h_hbm.at[colb], cb, sem)
        pltpu.make_async_copy(h_hbm.at[rowb], hb, sem).wait()
        pltpu.make_async_copy(h_hbm.at[colb], cb, sem).wait()

        def mul(i, _):
            for j in range(H // 16):
                hb[i, pl.ds(j * 16, 16)] = (
                    hb[i, pl.ds(j * 16, 16)] * cb[i, pl.ds(j * 16, 16)])
            return 0

        lax.fori_loop(0, BE, mul, 0)
        pltpu.sync_copy(hb, out_hbm.at[pl.ds(base, BE)])
        return 0

    lax.fori_loop(0, NB, blk, 0)


BN = 2000
BEF = 2000


def _node_body(at_ref, r_ref, p_ref, aemb_ref, afw_ref, lin0_ref,
               z_ref, hl0_ref):
    at = at_ref[...]
    iot = lax.broadcasted_iota(jnp.int32, (BN, 100), 1)
    onehot = (at == iot).astype(jnp.float32)
    a = jnp.dot(onehot, aemb_ref[...], preferred_element_type=jnp.float32)
    afr = jnp.dot(r_ref[...], afw_ref[...], preferred_element_type=jnp.float32)
    afp = jnp.dot(p_ref[...], afw_ref[...], preferred_element_type=jnp.float32)
    z = jnp.concatenate([a + afr, afp - afr], axis=1)
    z_ref[...] = z
    hl0_ref[...] = jnp.dot(z, lin0_ref[...], preferred_element_type=jnp.float32)


def _edge_dense_body(dsq_ref, bt_ref, mw1_ref, mb1_ref, mw2_ref, mb2_ref,
                     bemb_ref, cw1_ref, cb1_ref, cw2_ref, cb2_ref,
                     fw1_ref, fb1_ref, fw2_ref, fb2_ref,
                     el_ref, ea_ref, wf0_ref, wf1_ref, wf2_ref):
    ssq = jnp.sum(dsq_ref[...], axis=1, keepdims=True)
    el = jnp.sqrt(ssq + 1e-12)
    el_ref[...] = el
    g = jax.nn.relu(el * mw1_ref[...] + mb1_ref[...])
    g = jnp.dot(g, mw2_ref[...], preferred_element_type=jnp.float32) + mb2_ref[...]
    bt = bt_ref[...]
    iot = lax.broadcasted_iota(jnp.int32, (BEF, 100), 1)
    onehot = (bt == iot).astype(jnp.float32)
    be = jnp.dot(onehot, bemb_ref[...], preferred_element_type=jnp.float32)
    ear = g * be
    cat = jnp.concatenate([ear, ear], axis=1)
    ea = jax.nn.relu(
        jnp.dot(cat, cw1_ref[...], preferred_element_type=jnp.float32)
        + cb1_ref[...])
    ea = jnp.dot(ea, cw2_ref[...], preferred_element_type=jnp.float32) + cb2_ref[...]
    ea_ref[...] = ea
    for l, wf_ref in enumerate((wf0_ref, wf1_ref, wf2_ref)):
        wf = jax.nn.relu(
            jnp.dot(ea, fw1_ref[l], preferred_element_type=jnp.float32)
            + fb1_ref[l])
        wf_ref[...] = (
            jnp.dot(wf, fw2_ref[l], preferred_element_type=jnp.float32)
            + fb2_ref[l])


def _hupd_body(h_ref, a0_ref, a1_ref, ow_ref, ob_ref, lin_ref,
               h_out_ref, hl_out_ref):
    agg = jax.nn.relu(a0_ref[...] + a1_ref[...])
    hn = h_ref[...] + jnp.dot(
        agg, ow_ref[...], preferred_element_type=jnp.float32) + ob_ref[...]
    h_out_ref[...] = hn
    if hl_out_ref is not None:
        hl_out_ref[...] = jnp.dot(
            hn, lin_ref[...], preferred_element_type=jnp.float32)


def _hupd_last_body(h_ref, a0_ref, a1_ref, ow_ref, ob_ref, h_out_ref):
    _hupd_body(h_ref, a0_ref, a1_ref, ow_ref, ob_ref, None, h_out_ref, None)


def _pairmlp_body(hh_ref, ea_ref, w1_ref, b1_ref, w2_ref, b2_ref,
                  w3_ref, b3_ref, out_ref):
    x = jnp.concatenate([hh_ref[...], ea_ref[...]], axis=1)
    m = jax.nn.relu(
        jnp.dot(x, w1_ref[...], preferred_element_type=jnp.float32)
        + b1_ref[...])
    m = jax.nn.relu(
        jnp.dot(m, w2_ref[...], preferred_element_type=jnp.float32)
        + b2_ref[...])
    out_ref[...] = (
        jnp.dot(m, w3_ref[...], preferred_element_type=jnp.float32)
        + b3_ref[...])


def _full(shape):
    return pl.BlockSpec(shape, lambda i: tuple(0 for _ in shape))


def _rows(b, cols):
    return pl.BlockSpec((b, cols), lambda i: (i, 0))


def kernel(atom_type, r_feat, p_feat, pos, bond_index, bond_type, batch,
           time_step, atom_emb, atom_feat_W, bond_emb,
           edge_mlp_W1, edge_mlp_b1, edge_mlp_W2, edge_mlp_b2,
           edge_cat_W1, edge_cat_b1, edge_cat_W2, edge_cat_b2,
           filt_W1, filt_b1, filt_W2, filt_b2, lin_W, out_W, out_b,
           g_W1, g_b1, g_W2, g_b2, g_W3, g_b3):
    row = bond_index[0]
    col = bond_index[1]
    at2 = atom_type.astype(jnp.int32).reshape(N, 1)
    bt2 = bond_type.astype(jnp.int32).reshape(E, 1)
    row = row.astype(jnp.int32)
    col = col.astype(jnp.int32)

    z, hl = pl.pallas_call(
        _node_body,
        grid=(N // BN,),
        in_specs=[_rows(BN, 1), _rows(BN, FEAT), _rows(BN, FEAT),
                  _full((100, HALF)), _full((FEAT, HALF)), _full((H, H))],
        out_specs=[_rows(BN, H), _rows(BN, H)],
        out_shape=[jax.ShapeDtypeStruct((N, H), jnp.float32),
                   jax.ShapeDtypeStruct((N, H), jnp.float32)],
    )(at2, r_feat, p_feat, atom_emb, atom_feat_W, lin_W[0])

    posp = jnp.pad(pos, ((0, 0), (0, PP - 3)))
    dsq = _sc_geom(posp, row, col)

    el, ea, wf0, wf1, wf2 = pl.pallas_call(
        _edge_dense_body,
        grid=(E // BEF,),
        in_specs=[_rows(BEF, PP), _rows(BEF, 1),
                  _full((1, H)), _full((1, H)), _full((H, H)), _full((1, H)),
                  _full((100, H)),
                  _full((2 * H, H)), _full((1, H)), _full((H, H)), _full((1, H)),
                  _full((L, H, H)), _full((L, 1, H)),
                  _full((L, H, H)), _full((L, 1, H))],
        out_specs=[_rows(BEF, 1), _rows(BEF, H), _rows(BEF, H),
                   _rows(BEF, H), _rows(BEF, H)],
        out_shape=[jax.ShapeDtypeStruct((E, 1), jnp.float32),
                   jax.ShapeDtypeStruct((E, H), jnp.float32),
                   jax.ShapeDtypeStruct((E, H), jnp.float32),
                   jax.ShapeDtypeStruct((E, H), jnp.float32),
                   jax.ShapeDtypeStruct((E, H), jnp.float32)],
    )(dsq, bt2,
      edge_mlp_W1, edge_mlp_b1.reshape(1, H), edge_mlp_W2,
      edge_mlp_b2.reshape(1, H), bond_emb,
      edge_cat_W1, edge_cat_b1.reshape(1, H), edge_cat_W2,
      edge_cat_b2.reshape(1, H),
      filt_W1, filt_b1.reshape(L, 1, H), filt_W2, filt_b2.reshape(L, 1, H))

    wfs = (wf0, wf1, wf2)

    h = z
    for l in range(L):
        parts = _sc_msg(hl, wfs[l], row, col)
        a0 = parts[:N]
        a1 = parts[N:]
        if l < L - 1:
            h, hl = pl.pallas_call(
                _hupd_body,
                grid=(N // BN,),
                in_specs=[_rows(BN, H), _rows(BN, H), _rows(BN, H),
                          _full((H, H)), _full((1, H)), _full((H, H))],
                out_specs=[_rows(BN, H), _rows(BN, H)],
                out_shape=[jax.ShapeDtypeStruct((N, H), jnp.float32),
                           jax.ShapeDtypeStruct((N, H), jnp.float32)],
            )(h, a0, a1, out_W[l], out_b[l].reshape(1, H), lin_W[l + 1])
        else:
            h = pl.pallas_call(
                _hupd_last_body,
                grid=(N // BN,),
                in_specs=[_rows(BN, H), _rows(BN, H), _rows(BN, H),
                          _full((H, H)), _full((1, H))],
                out_specs=_rows(BN, H),
                out_shape=jax.ShapeDtypeStruct((N, H), jnp.float32),
            )(h, a0, a1, out_W[l], out_b[l].reshape(1, H))

    hh = _sc_pair(h, row, col)

    edge_inv = pl.pallas_call(
        _pairmlp_body,
        grid=(E // BEF,),
        in_specs=[_rows(BEF, H), _rows(BEF, H),
                  _full((2 * H, H)), _full((1, H)),
                  _full((H, HALF)), _full((1, HALF)),
                  _full((HALF, 1)), _full((1, 1))],
        out_specs=_rows(BEF, 1),
        out_shape=jax.ShapeDtypeStruct((E, 1), jnp.float32),
    )(hh, ea, g_W1, g_b1.reshape(1, H), g_W2, g_b2.reshape(1, HALF),
      g_W3, g_b3.reshape(1, 1))

    return (edge_inv, bond_index, el)


# speedup vs baseline: 3.1032x; 1.1478x over previous
import functools

import jax
import jax.numpy as jnp
from jax import lax
from jax.experimental import pallas as pl
from jax.experimental.pallas import tpu as pltpu
from jax.experimental.pallas import tpu_sc as plsc

N = 10000
E = 320000
H = 64
HALF = 32
FEAT = 128
L = 3

NC = 2
NS = 16
NW = NC * NS
EC = E // NW
BE = 80
NB = EC // BE
G = 5
GB = G * BE
NGRP = NB // G
NR = N // NS
RB = 125

_mesh = plsc.VectorSubcoreMesh(core_axis_name="c", subcore_axis_name="s")


PP = 16


@functools.partial(
    pl.kernel,
    mesh=_mesh,
    compiler_params=pltpu.CompilerParams(use_tc_tiling_on_sc=False),
    out_type=jax.ShapeDtypeStruct((E, PP), jnp.float32),
    scratch_types=[
        pltpu.VMEM((NB, BE), jnp.int32),
        pltpu.VMEM((NB, BE), jnp.int32),
        pltpu.VMEM((2, GB, PP), jnp.float32),
        pltpu.VMEM((2, GB, PP), jnp.float32),
        pltpu.SemaphoreType.DMA((G,)),
        pltpu.SemaphoreType.DMA((G,)),
        pltpu.SemaphoreType.DMA,
    ],
)
def _sc_geom(posp_hbm, row3_hbm, col3_hbm, out_hbm,
             rowbig, colbig, pr2, pc2, gsem, csem, wsem):
    c = lax.axis_index("c")
    s = lax.axis_index("s")
    w = c * NS + s
    pltpu.sync_copy(row3_hbm.at[w], rowbig)
    pltpu.sync_copy(col3_hbm.at[w], colbig)

    def issue(g, q):
        for b in range(G):
            k = g * G + b
            dst = pl.ds(b * BE, BE)
            pltpu.async_copy(posp_hbm.at[rowbig.at[k]], pr2.at[q, dst], gsem.at[b])
            pltpu.async_copy(posp_hbm.at[colbig.at[k]], pc2.at[q, dst], csem.at[b])

    def wait_g(g, q):
        for b in range(G):
            k = g * G + b
            dst = pl.ds(b * BE, BE)
            pltpu.make_async_copy(posp_hbm.at[rowbig.at[k]], pr2.at[q, dst],
                                  gsem.at[b]).wait()
            pltpu.make_async_copy(posp_hbm.at[colbig.at[k]], pc2.at[q, dst],
                                  csem.at[b]).wait()

    def wout(g, q):
        base = w * EC + g * GB
        return pltpu.make_async_copy(pr2.at[q], out_hbm.at[pl.ds(base, GB)], wsem)

    issue(0, 0)

    def grp(g, _):
        q = lax.rem(g, 2)
        wait_g(g, q)

        def body(i, _):
            d = pr2[q, i] - pc2[q, i]
            pr2[q, i] = d * d
            return 0

        lax.fori_loop(0, GB, body, 0)

        @pl.when(g > 0)
        def _():
            wout(g - 1, 1 - q).wait()

        wout(g, q).start()

        @pl.when(g < NGRP - 1)
        def _():
            issue(g + 1, 1 - q)

        return 0

    lax.fori_loop(0, NGRP, grp, 0)
    wout(NGRP - 1, lax.rem(NGRP - 1, 2)).wait()


@functools.partial(
    pl.kernel,
    mesh=_mesh,
    compiler_params=pltpu.CompilerParams(use_tc_tiling_on_sc=False),
    out_type=jax.ShapeDtypeStruct((2 * N, H), jnp.float32),
    scratch_types=[
        pltpu.VMEM_SHARED((N, H), jnp.float32),
        pltpu.VMEM((2, G, BE), jnp.int32),
        pltpu.VMEM((2, G, BE), jnp.int32),
        pltpu.VMEM((2, GB, H), jnp.float32),
        pltpu.VMEM((GB, H), jnp.float32),
        pltpu.SemaphoreType.DMA((G,)),
        pltpu.SemaphoreType.DMA((G,)),
    ],
)
def _sc_msg(hl_hbm, wf_hbm, row3_hbm, col3_hbm, out_hbm,
            acc, idxr2, idxc2, hb2, wfc, gsem, ssem):
    c = lax.axis_index("c")
    s = lax.axis_index("s")
    w = c * NS + s

    def idxload(g, q):
        pltpu.sync_copy(row3_hbm.at[w, pl.ds(g * G, G)], idxr2.at[q])
        pltpu.sync_copy(col3_hbm.at[w, pl.ds(g * G, G)], idxc2.at[q])

    def zero(i, _):
        for j in range(H // 16):
            wfc[i, pl.ds(j * 16, 16)] = jnp.zeros((16,), jnp.float32)
        return 0

    lax.fori_loop(0, RB, zero, 0)
    for jj in range(NR // RB):
        pltpu.sync_copy(wfc.at[pl.ds(0, RB)],
                        acc.at[pl.ds(s * NR + jj * RB, RB)])
    plsc.subcore_barrier()

    def gissue(q):
        for b in range(G):
            pltpu.async_copy(hl_hbm.at[idxr2.at[q, b]],
                             hb2.at[q, pl.ds(b * BE, BE)], gsem.at[b])

    def gwait(q):
        for b in range(G):
            pltpu.make_async_copy(hl_hbm.at[idxr2.at[q, b]],
                                  hb2.at[q, pl.ds(b * BE, BE)],
                                  gsem.at[b]).wait()

    def sdesc(q, b):
        return pltpu.make_async_copy(hb2.at[q, pl.ds(b * BE, BE)],
                                     acc.at[idxc2.at[q, b]],
                                     ssem.at[b])

    def wfload(g):
        pltpu.sync_copy(wf_hbm.at[pl.ds(w * EC + g * GB, GB)], wfc)

    idxload(0, 0)
    gissue(0)
    wfload(0)

    def grp(g, _):
        q = lax.rem(g, 2)
        gwait(q)

        def mul(i, _):
            for j in range(H // 16):
                sl = pl.ds(j * 16, 16)
                hb2[q, i, sl] = hb2[q, i, sl] * wfc[i, sl]
            return 0

        lax.fori_loop(0, GB, mul, 0)

        @pl.when(g > 0)
        def _():
            for b in range(G):
                sdesc(1 - q, b).wait()

        for b in range(G):
            sdesc(q, b).start(add=True)

        @pl.when(g < NGRP - 1)
        def _():
            idxload(g + 1, 1 - q)
            gissue(1 - q)
            wfload(g + 1)

        return 0

    lax.fori_loop(0, NGRP, grp, 0)
    for b in range(G):
        sdesc(lax.rem(NGRP - 1, 2), b).wait()
    plsc.subcore_barrier()
    for jj in range(NR // RB):
        off = s * NR + jj * RB
        pltpu.sync_copy(acc.at[pl.ds(off, RB)], wfc.at[pl.ds(0, RB)])
        pltpu.sync_copy(wfc.at[pl.ds(0, RB)], out_hbm.at[pl.ds(c * N + off, RB)])


@functools.partial(
    pl.kernel,
    mesh=_mesh,
    compiler_params=pltpu.CompilerParams(use_tc_tiling_on_sc=False),
    out_type=jax.ShapeDtypeStruct((E, H), jnp.float32),
    scratch_types=[
        pltpu.VMEM((NB, BE), jnp.int32),
        pltpu.VMEM((NB, BE), jnp.int32),
        pltpu.VMEM((2, GB, H), jnp.float32),
        pltpu.VMEM((GB, H), jnp.float32),
        pltpu.SemaphoreType.DMA((G,)),
        pltpu.SemaphoreType.DMA((G,)),
        pltpu.SemaphoreType.DMA,
    ],
)
def _sc_pair(h_hbm, row3_hbm, col3_hbm, out_hbm,
             rowbig, colbig, hr2, hc, gsem, csem, wsem):
    c = lax.axis_index("c")
    s = lax.axis_index("s")
    w = c * NS + s
    pltpu.sync_copy(row3_hbm.at[w], rowbig)
    pltpu.sync_copy(col3_hbm.at[w], colbig)

    def rissue(g, q):
        for b in range(G):
            pltpu.async_copy(h_hbm.at[rowbig.at[g * G + b]],
                             hr2.at[q, pl.ds(b * BE, BE)], gsem.at[b])

    def rwait(g, q):
        for b in range(G):
            pltpu.make_async_copy(h_hbm.at[rowbig.at[g * G + b]],
                                  hr2.at[q, pl.ds(b * BE, BE)],
                                  gsem.at[b]).wait()

    def cissue(g):
        for b in range(G):
            pltpu.async_copy(h_hbm.at[colbig.at[g * G + b]],
                             hc.at[pl.ds(b * BE, BE)], csem.at[b])

    def cwait(g):
        for b in range(G):
            pltpu.make_async_copy(h_hbm.at[colbig.at[g * G + b]],
                                  hc.at[pl.ds(b * BE, BE)], csem.at[b]).wait()

    def wout(g, q):
        base = w * EC + g * GB
        return pltpu.make_async_copy(hr2.at[q], out_hbm.at[pl.ds(base, GB)], wsem)

    rissue(0, 0)
    cissue(0)

    def grp(g, _):
        q = lax.rem(g, 2)
        rwait(g, q)
        cwait(g)

        def mul(i, _):
            for j in range(H // 16):
                sl = pl.ds(j * 16, 16)
                hr2[q, i, sl] = hr2[q, i, sl] * hc[i, sl]
            return 0

        lax.fori_loop(0, GB, mul, 0)

        @pl.when(g > 0)
        def _():
            wout(g - 1, 1 - q).wait()

        wout(g, q).start()

        @pl.when(g < NGRP - 1)
        def _():
            rissue(g + 1, 1 - q)
            cissue(g + 1)

        return 0

    lax.fori_loop(0, NGRP, grp, 0)
    wout(NGRP - 1, lax.rem(NGRP - 1, 2)).wait()


BN = 2000
BEF = 2000


def _node_body(at_ref, r_ref, p_ref, aemb_ref, afw_ref, lin0_ref,
               z_ref, hl0_ref):
    at = at_ref[...]
    iot = lax.broadcasted_iota(jnp.int32, (BN, 100), 1)
    onehot = (at == iot).astype(jnp.float32)
    a = jnp.dot(onehot, aemb_ref[...], preferred_element_type=jnp.float32)
    afr = jnp.dot(r_ref[...], afw_ref[...], preferred_element_type=jnp.float32)
    afp = jnp.dot(p_ref[...], afw_ref[...], preferred_element_type=jnp.float32)
    z = jnp.concatenate([a + afr, afp - afr], axis=1)
    z_ref[...] = z
    hl0_ref[...] = jnp.dot(z, lin0_ref[...], preferred_element_type=jnp.float32)


def _edge_dense_body(dsq_ref, bt_ref, mw1_ref, mb1_ref, mw2_ref, mb2_ref,
                     bemb_ref, cw1_ref, cb1_ref, cw2_ref, cb2_ref,
                     fw1_ref, fb1_ref, fw2_ref, fb2_ref,
                     el_ref, ea_ref, wf0_ref, wf1_ref, wf2_ref):
    ssq = jnp.sum(dsq_ref[...], axis=1, keepdims=True)
    el = jnp.sqrt(ssq + 1e-12)
    el_ref[...] = el
    g = jax.nn.relu(el * mw1_ref[...] + mb1_ref[...])
    g = jnp.dot(g, mw2_ref[...], preferred_element_type=jnp.float32) + mb2_ref[...]
    bt = bt_ref[...]
    iot = lax.broadcasted_iota(jnp.int32, (BEF, 100), 1)
    onehot = (bt == iot).astype(jnp.float32)
    be = jnp.dot(onehot, bemb_ref[...], preferred_element_type=jnp.float32)
    ear = g * be
    cat = jnp.concatenate([ear, ear], axis=1)
    ea = jax.nn.relu(
        jnp.dot(cat, cw1_ref[...], preferred_element_type=jnp.float32)
        + cb1_ref[...])
    ea = jnp.dot(ea, cw2_ref[...], preferred_element_type=jnp.float32) + cb2_ref[...]
    ea_ref[...] = ea
    for l, wf_ref in enumerate((wf0_ref, wf1_ref, wf2_ref)):
        wf = jax.nn.relu(
            jnp.dot(ea, fw1_ref[l], preferred_element_type=jnp.float32)
            + fb1_ref[l])
        wf_ref[...] = (
            jnp.dot(wf, fw2_ref[l], preferred_element_type=jnp.float32)
            + fb2_ref[l])


def _hupd_body(h_ref, a0_ref, a1_ref, ow_ref, ob_ref, lin_ref,
               h_out_ref, hl_out_ref):
    agg = jax.nn.relu(a0_ref[...] + a1_ref[...])
    hn = h_ref[...] + jnp.dot(
        agg, ow_ref[...], preferred_element_type=jnp.float32) + ob_ref[...]
    h_out_ref[...] = hn
    if hl_out_ref is not None:
        hl_out_ref[...] = jnp.dot(
            hn, lin_ref[...], preferred_element_type=jnp.float32)


def _hupd_last_body(h_ref, a0_ref, a1_ref, ow_ref, ob_ref, h_out_ref):
    _hupd_body(h_ref, a0_ref, a1_ref, ow_ref, ob_ref, None, h_out_ref, None)


def _pairmlp_body(hh_ref, ea_ref, w1_ref, b1_ref, w2_ref, b2_ref,
                  w3_ref, b3_ref, out_ref):
    x = jnp.concatenate([hh_ref[...], ea_ref[...]], axis=1)
    m = jax.nn.relu(
        jnp.dot(x, w1_ref[...], preferred_element_type=jnp.float32)
        + b1_ref[...])
    m = jax.nn.relu(
        jnp.dot(m, w2_ref[...], preferred_element_type=jnp.float32)
        + b2_ref[...])
    out_ref[...] = (
        jnp.dot(m, w3_ref[...], preferred_element_type=jnp.float32)
        + b3_ref[...])


def _full(shape):
    return pl.BlockSpec(shape, lambda i: tuple(0 for _ in shape))


def _rows(b, cols):
    return pl.BlockSpec((b, cols), lambda i: (i, 0))


def kernel(atom_type, r_feat, p_feat, pos, bond_index, bond_type, batch,
           time_step, atom_emb, atom_feat_W, bond_emb,
           edge_mlp_W1, edge_mlp_b1, edge_mlp_W2, edge_mlp_b2,
           edge_cat_W1, edge_cat_b1, edge_cat_W2, edge_cat_b2,
           filt_W1, filt_b1, filt_W2, filt_b2, lin_W, out_W, out_b,
           g_W1, g_b1, g_W2, g_b2, g_W3, g_b3):
    row = bond_index[0]
    col = bond_index[1]
    at2 = atom_type.astype(jnp.int32).reshape(N, 1)
    bt2 = bond_type.astype(jnp.int32).reshape(E, 1)
    row = row.astype(jnp.int32)
    col = col.astype(jnp.int32)
    row3 = row.reshape(NW, NB, BE)
    col3 = col.reshape(NW, NB, BE)

    z, hl = pl.pallas_call(
        _node_body,
        grid=(N // BN,),
        in_specs=[_rows(BN, 1), _rows(BN, FEAT), _rows(BN, FEAT),
                  _full((100, HALF)), _full((FEAT, HALF)), _full((H, H))],
        out_specs=[_rows(BN, H), _rows(BN, H)],
        out_shape=[jax.ShapeDtypeStruct((N, H), jnp.float32),
                   jax.ShapeDtypeStruct((N, H), jnp.float32)],
    )(at2, r_feat, p_feat, atom_emb, atom_feat_W, lin_W[0])

    posp = jnp.pad(pos, ((0, 0), (0, PP - 3)))
    dsq = _sc_geom(posp, row3, col3)

    el, ea, wf0, wf1, wf2 = pl.pallas_call(
        _edge_dense_body,
        grid=(E // BEF,),
        in_specs=[_rows(BEF, PP), _rows(BEF, 1),
                  _full((1, H)), _full((1, H)), _full((H, H)), _full((1, H)),
                  _full((100, H)),
                  _full((2 * H, H)), _full((1, H)), _full((H, H)), _full((1, H)),
                  _full((L, H, H)), _full((L, 1, H)),
                  _full((L, H, H)), _full((L, 1, H))],
        out_specs=[_rows(BEF, 1), _rows(BEF, H), _rows(BEF, H),
                   _rows(BEF, H), _rows(BEF, H)],
        out_shape=[jax.ShapeDtypeStruct((E, 1), jnp.float32),
                   jax.ShapeDtypeStruct((E, H), jnp.float32),
                   jax.ShapeDtypeStruct((E, H), jnp.float32),
                   jax.ShapeDtypeStruct((E, H), jnp.float32),
                   jax.ShapeDtypeStruct((E, H), jnp.float32)],
    )(dsq, bt2,
      edge_mlp_W1, edge_mlp_b1.reshape(1, H), edge_mlp_W2,
      edge_mlp_b2.reshape(1, H), bond_emb,
      edge_cat_W1, edge_cat_b1.reshape(1, H), edge_cat_W2,
      edge_cat_b2.reshape(1, H),
      filt_W1, filt_b1.reshape(L, 1, H), filt_W2, filt_b2.reshape(L, 1, H))

    wfs = (wf0, wf1, wf2)

    h = z
    for l in range(L):
        parts = _sc_msg(hl, wfs[l], row3, col3)
        a0 = parts[:N]
        a1 = parts[N:]
        if l < L - 1:
            h, hl = pl.pallas_call(
                _hupd_body,
                grid=(N // BN,),
                in_specs=[_rows(BN, H), _rows(BN, H), _rows(BN, H),
                          _full((H, H)), _full((1, H)), _full((H, H))],
                out_specs=[_rows(BN, H), _rows(BN, H)],
                out_shape=[jax.ShapeDtypeStruct((N, H), jnp.float32),
                           jax.ShapeDtypeStruct((N, H), jnp.float32)],
            )(h, a0, a1, out_W[l], out_b[l].reshape(1, H), lin_W[l + 1])
        else:
            h = pl.pallas_call(
                _hupd_last_body,
                grid=(N // BN,),
                in_specs=[_rows(BN, H), _rows(BN, H), _rows(BN, H),
                          _full((H, H)), _full((1, H))],
                out_specs=_rows(BN, H),
                out_shape=jax.ShapeDtypeStruct((N, H), jnp.float32),
            )(h, a0, a1, out_W[l], out_b[l].reshape(1, H))

    hh = _sc_pair(h, row3, col3)

    edge_inv = pl.pallas_call(
        _pairmlp_body,
        grid=(E // BEF,),
        in_specs=[_rows(BEF, H), _rows(BEF, H),
                  _full((2 * H, H)), _full((1, H)),
                  _full((H, HALF)), _full((1, HALF)),
                  _full((HALF, 1)), _full((1, 1))],
        out_specs=_rows(BEF, 1),
        out_shape=jax.ShapeDtypeStruct((E, 1), jnp.float32),
    )(hh, ea, g_W1, g_b1.reshape(1, H), g_W2, g_b2.reshape(1, HALF),
      g_W3, g_b3.reshape(1, 1))

    return (edge_inv, bond_index, el)


# speedup vs baseline: 4.0851x; 1.3164x over previous
import functools

import jax
import jax.numpy as jnp
from jax import lax
from jax.experimental import pallas as pl
from jax.experimental.pallas import tpu as pltpu
from jax.experimental.pallas import tpu_sc as plsc

N = 10000
E = 320000
H = 64
HALF = 32
FEAT = 128
L = 3

NC = 2
NS = 16
NW = NC * NS
EC = E // NW
BE = 80
NB = EC // BE
G = 5
GB = G * BE
NGRP = NB // G
NR = N // NS
RB = 125

_mesh = plsc.VectorSubcoreMesh(core_axis_name="c", subcore_axis_name="s")


PP = 16


@functools.partial(
    pl.kernel,
    mesh=_mesh,
    compiler_params=pltpu.CompilerParams(use_tc_tiling_on_sc=False),
    out_type=jax.ShapeDtypeStruct((E, PP), jnp.float32),
    scratch_types=[
        pltpu.VMEM((NB, BE), jnp.int32),
        pltpu.VMEM((NB, BE), jnp.int32),
        pltpu.VMEM((2, GB, PP), jnp.float32),
        pltpu.VMEM((2, GB, PP), jnp.float32),
        pltpu.SemaphoreType.DMA((G,)),
        pltpu.SemaphoreType.DMA((G,)),
        pltpu.SemaphoreType.DMA,
    ],
)
def _sc_geom(posp_hbm, row3_hbm, col3_hbm, out_hbm,
             rowbig, colbig, pr2, pc2, gsem, csem, wsem):
    c = lax.axis_index("c")
    s = lax.axis_index("s")
    w = c * NS + s
    pltpu.sync_copy(row3_hbm.at[w], rowbig)
    pltpu.sync_copy(col3_hbm.at[w], colbig)

    def issue(g, q):
        for b in range(G):
            k = g * G + b
            dst = pl.ds(b * BE, BE)
            pltpu.async_copy(posp_hbm.at[rowbig.at[k]], pr2.at[q, dst], gsem.at[b])
            pltpu.async_copy(posp_hbm.at[colbig.at[k]], pc2.at[q, dst], csem.at[b])

    def wait_g(g, q):
        for b in range(G):
            k = g * G + b
            dst = pl.ds(b * BE, BE)
            pltpu.make_async_copy(posp_hbm.at[rowbig.at[k]], pr2.at[q, dst],
                                  gsem.at[b]).wait()
            pltpu.make_async_copy(posp_hbm.at[colbig.at[k]], pc2.at[q, dst],
                                  csem.at[b]).wait()

    def wout(g, q):
        base = w * EC + g * GB
        return pltpu.make_async_copy(pr2.at[q], out_hbm.at[pl.ds(base, GB)], wsem)

    issue(0, 0)

    def grp(g, _):
        q = lax.rem(g, 2)
        wait_g(g, q)

        @plsc.parallel_loop(0, GB, unroll=8)
        def _(i):
            d = pr2[q, i] - pc2[q, i]
            pr2[q, i] = d * d

        @pl.when(g > 0)
        def _():
            wout(g - 1, 1 - q).wait()

        wout(g, q).start()

        @pl.when(g < NGRP - 1)
        def _():
            issue(g + 1, 1 - q)

        return 0

    lax.fori_loop(0, NGRP, grp, 0)
    wout(NGRP - 1, lax.rem(NGRP - 1, 2)).wait()


@functools.partial(
    pl.kernel,
    mesh=_mesh,
    compiler_params=pltpu.CompilerParams(use_tc_tiling_on_sc=False),
    out_type=jax.ShapeDtypeStruct((2 * N, H), jnp.float32),
    scratch_types=[
        pltpu.VMEM_SHARED((N, H), jnp.float32),
        pltpu.VMEM((2, G, BE), jnp.int32),
        pltpu.VMEM((2, G, BE), jnp.int32),
        pltpu.VMEM((2, GB, H), jnp.float32),
        pltpu.VMEM((GB, H), jnp.float32),
        pltpu.SemaphoreType.DMA((G,)),
        pltpu.SemaphoreType.DMA((G,)),
    ],
)
def _sc_msg(hl_hbm, wf_hbm, row3_hbm, col3_hbm, out_hbm,
            acc, idxr2, idxc2, hb2, wfc, gsem, ssem):
    c = lax.axis_index("c")
    s = lax.axis_index("s")
    w = c * NS + s

    def idxload(g, q):
        pltpu.sync_copy(row3_hbm.at[w, pl.ds(g * G, G)], idxr2.at[q])
        pltpu.sync_copy(col3_hbm.at[w, pl.ds(g * G, G)], idxc2.at[q])

    def zero(i, _):
        for j in range(H // 16):
            wfc[i, pl.ds(j * 16, 16)] = jnp.zeros((16,), jnp.float32)
        return 0

    lax.fori_loop(0, RB, zero, 0)
    for jj in range(NR // RB):
        pltpu.sync_copy(wfc.at[pl.ds(0, RB)],
                        acc.at[pl.ds(s * NR + jj * RB, RB)])
    plsc.subcore_barrier()

    def gissue(q):
        for b in range(G):
            pltpu.async_copy(hl_hbm.at[idxr2.at[q, b]],
                             hb2.at[q, pl.ds(b * BE, BE)], gsem.at[b])

    def gwait(q):
        for b in range(G):
            pltpu.make_async_copy(hl_hbm.at[idxr2.at[q, b]],
                                  hb2.at[q, pl.ds(b * BE, BE)],
                                  gsem.at[b]).wait()

    def sdesc(q, b):
        return pltpu.make_async_copy(hb2.at[q, pl.ds(b * BE, BE)],
                                     acc.at[idxc2.at[q, b]],
                                     ssem.at[b])

    def wfload(g):
        pltpu.sync_copy(wf_hbm.at[pl.ds(w * EC + g * GB, GB)], wfc)

    idxload(0, 0)
    gissue(0)
    wfload(0)

    def grp(g, _):
        q = lax.rem(g, 2)
        gwait(q)

        @plsc.parallel_loop(0, GB, unroll=4)
        def _(i):
            for j in range(H // 16):
                sl = pl.ds(j * 16, 16)
                hb2[q, i, sl] = hb2[q, i, sl] * wfc[i, sl]

        @pl.when(g > 0)
        def _():
            for b in range(G):
                sdesc(1 - q, b).wait()

        for b in range(G):
            sdesc(q, b).start(add=True)

        @pl.when(g < NGRP - 1)
        def _():
            idxload(g + 1, 1 - q)
            gissue(1 - q)
            wfload(g + 1)

        return 0

    lax.fori_loop(0, NGRP, grp, 0)
    for b in range(G):
        sdesc(lax.rem(NGRP - 1, 2), b).wait()
    plsc.subcore_barrier()
    for jj in range(NR // RB):
        off = s * NR + jj * RB
        pltpu.sync_copy(acc.at[pl.ds(off, RB)], wfc.at[pl.ds(0, RB)])
        pltpu.sync_copy(wfc.at[pl.ds(0, RB)], out_hbm.at[pl.ds(c * N + off, RB)])


@functools.partial(
    pl.kernel,
    mesh=_mesh,
    compiler_params=pltpu.CompilerParams(use_tc_tiling_on_sc=False),
    out_type=jax.ShapeDtypeStruct((E, H), jnp.float32),
    scratch_types=[
        pltpu.VMEM((NB, BE), jnp.int32),
        pltpu.VMEM((NB, BE), jnp.int32),
        pltpu.VMEM((2, GB, H), jnp.float32),
        pltpu.VMEM((GB, H), jnp.float32),
        pltpu.SemaphoreType.DMA((G,)),
        pltpu.SemaphoreType.DMA((G,)),
        pltpu.SemaphoreType.DMA,
    ],
)
def _sc_pair(h_hbm, row3_hbm, col3_hbm, out_hbm,
             rowbig, colbig, hr2, hc, gsem, csem, wsem):
    c = lax.axis_index("c")
    s = lax.axis_index("s")
    w = c * NS + s
    pltpu.sync_copy(row3_hbm.at[w], rowbig)
    pltpu.sync_copy(col3_hbm.at[w], colbig)

    def rissue(g, q):
        for b in range(G):
            pltpu.async_copy(h_hbm.at[rowbig.at[g * G + b]],
                             hr2.at[q, pl.ds(b * BE, BE)], gsem.at[b])

    def rwait(g, q):
        for b in range(G):
            pltpu.make_async_copy(h_hbm.at[rowbig.at[g * G + b]],
                                  hr2.at[q, pl.ds(b * BE, BE)],
                                  gsem.at[b]).wait()

    def cissue(g):
        for b in range(G):
            pltpu.async_copy(h_hbm.at[colbig.at[g * G + b]],
                             hc.at[pl.ds(b * BE, BE)], csem.at[b])

    def cwait(g):
        for b in range(G):
            pltpu.make_async_copy(h_hbm.at[colbig.at[g * G + b]],
                                  hc.at[pl.ds(b * BE, BE)], csem.at[b]).wait()

    def wout(g, q):
        base = w * EC + g * GB
        return pltpu.make_async_copy(hr2.at[q], out_hbm.at[pl.ds(base, GB)], wsem)

    rissue(0, 0)
    cissue(0)

    def grp(g, _):
        q = lax.rem(g, 2)
        rwait(g, q)
        cwait(g)

        @plsc.parallel_loop(0, GB, unroll=4)
        def _(i):
            for j in range(H // 16):
                sl = pl.ds(j * 16, 16)
                hr2[q, i, sl] = hr2[q, i, sl] * hc[i, sl]

        @pl.when(g > 0)
        def _():
            wout(g - 1, 1 - q).wait()

        wout(g, q).start()

        @pl.when(g < NGRP - 1)
        def _():
            rissue(g + 1, 1 - q)
            cissue(g + 1)

        return 0

    lax.fori_loop(0, NGRP, grp, 0)
    wout(NGRP - 1, lax.rem(NGRP - 1, 2)).wait()


BN = 2000
BEF = 2000


def _node_body(at_ref, r_ref, p_ref, aemb_ref, afw_ref, lin0_ref,
               z_ref, hl0_ref):
    at = at_ref[...]
    iot = lax.broadcasted_iota(jnp.int32, (BN, 100), 1)
    onehot = (at == iot).astype(jnp.float32)
    a = jnp.dot(onehot, aemb_ref[...], preferred_element_type=jnp.float32)
    afr = jnp.dot(r_ref[...], afw_ref[...], preferred_element_type=jnp.float32)
    afp = jnp.dot(p_ref[...], afw_ref[...], preferred_element_type=jnp.float32)
    z = jnp.concatenate([a + afr, afp - afr], axis=1)
    z_ref[...] = z
    hl0_ref[...] = jnp.dot(z, lin0_ref[...], preferred_element_type=jnp.float32)


def _edge_dense_body(dsq_ref, bt_ref, mw1_ref, mb1_ref, mw2_ref, mb2_ref,
                     bemb_ref, cw1_ref, cb1_ref, cw2_ref, cb2_ref,
                     fw1_ref, fb1_ref, fw2_ref, fb2_ref,
                     el_ref, ea_ref, wf0_ref, wf1_ref, wf2_ref):
    ssq = jnp.sum(dsq_ref[...], axis=1, keepdims=True)
    el = jnp.sqrt(ssq + 1e-12)
    el_ref[...] = el
    g = jax.nn.relu(el * mw1_ref[...] + mb1_ref[...])
    g = jnp.dot(g, mw2_ref[...], preferred_element_type=jnp.float32) + mb2_ref[...]
    bt = bt_ref[...]
    iot = lax.broadcasted_iota(jnp.int32, (BEF, 100), 1)
    onehot = (bt == iot).astype(jnp.float32)
    be = jnp.dot(onehot, bemb_ref[...], preferred_element_type=jnp.float32)
    ear = g * be
    cat = jnp.concatenate([ear, ear], axis=1)
    ea = jax.nn.relu(
        jnp.dot(cat, cw1_ref[...], preferred_element_type=jnp.float32)
        + cb1_ref[...])
    ea = jnp.dot(ea, cw2_ref[...], preferred_element_type=jnp.float32) + cb2_ref[...]
    ea_ref[...] = ea
    for l, wf_ref in enumerate((wf0_ref, wf1_ref, wf2_ref)):
        wf = jax.nn.relu(
            jnp.dot(ea, fw1_ref[l], preferred_element_type=jnp.float32)
            + fb1_ref[l])
        wf_ref[...] = (
            jnp.dot(wf, fw2_ref[l], preferred_element_type=jnp.float32)
            + fb2_ref[l])


def _hupd_body(h_ref, a0_ref, a1_ref, ow_ref, ob_ref, lin_ref,
               h_out_ref, hl_out_ref):
    agg = jax.nn.relu(a0_ref[...] + a1_ref[...])
    hn = h_ref[...] + jnp.dot(
        agg, ow_ref[...], preferred_element_type=jnp.float32) + ob_ref[...]
    h_out_ref[...] = hn
    if hl_out_ref is not None:
        hl_out_ref[...] = jnp.dot(
            hn, lin_ref[...], preferred_element_type=jnp.float32)


def _hupd_last_body(h_ref, a0_ref, a1_ref, ow_ref, ob_ref, h_out_ref):
    _hupd_body(h_ref, a0_ref, a1_ref, ow_ref, ob_ref, None, h_out_ref, None)


def _pairmlp_body(hh_ref, ea_ref, w1_ref, b1_ref, w2_ref, b2_ref,
                  w3_ref, b3_ref, out_ref):
    x = jnp.concatenate([hh_ref[...], ea_ref[...]], axis=1)
    m = jax.nn.relu(
        jnp.dot(x, w1_ref[...], preferred_element_type=jnp.float32)
        + b1_ref[...])
    m = jax.nn.relu(
        jnp.dot(m, w2_ref[...], preferred_element_type=jnp.float32)
        + b2_ref[...])
    out_ref[...] = (
        jnp.dot(m, w3_ref[...], preferred_element_type=jnp.float32)
        + b3_ref[...])


def _full(shape):
    return pl.BlockSpec(shape, lambda i: tuple(0 for _ in shape))


def _rows(b, cols):
    return pl.BlockSpec((b, cols), lambda i: (i, 0))


def kernel(atom_type, r_feat, p_feat, pos, bond_index, bond_type, batch,
           time_step, atom_emb, atom_feat_W, bond_emb,
           edge_mlp_W1, edge_mlp_b1, edge_mlp_W2, edge_mlp_b2,
           edge_cat_W1, edge_cat_b1, edge_cat_W2, edge_cat_b2,
           filt_W1, filt_b1, filt_W2, filt_b2, lin_W, out_W, out_b,
           g_W1, g_b1, g_W2, g_b2, g_W3, g_b3):
    row = bond_index[0]
    col = bond_index[1]
    at2 = atom_type.astype(jnp.int32).reshape(N, 1)
    bt2 = bond_type.astype(jnp.int32).reshape(E, 1)
    row = row.astype(jnp.int32)
    col = col.astype(jnp.int32)
    row3 = row.reshape(NW, NB, BE)
    col3 = col.reshape(NW, NB, BE)

    z, hl = pl.pallas_call(
        _node_body,
        grid=(N // BN,),
        in_specs=[_rows(BN, 1), _rows(BN, FEAT), _rows(BN, FEAT),
                  _full((100, HALF)), _full((FEAT, HALF)), _full((H, H))],
        out_specs=[_rows(BN, H), _rows(BN, H)],
        out_shape=[jax.ShapeDtypeStruct((N, H), jnp.float32),
                   jax.ShapeDtypeStruct((N, H), jnp.float32)],
    )(at2, r_feat, p_feat, atom_emb, atom_feat_W, lin_W[0])

    posp = jnp.pad(pos, ((0, 0), (0, PP - 3)))
    dsq = _sc_geom(posp, row3, col3)

    el, ea, wf0, wf1, wf2 = pl.pallas_call(
        _edge_dense_body,
        grid=(E // BEF,),
        in_specs=[_rows(BEF, PP), _rows(BEF, 1),
                  _full((1, H)), _full((1, H)), _full((H, H)), _full((1, H)),
                  _full((100, H)),
                  _full((2 * H, H)), _full((1, H)), _full((H, H)), _full((1, H)),
                  _full((L, H, H)), _full((L, 1, H)),
                  _full((L, H, H)), _full((L, 1, H))],
        out_specs=[_rows(BEF, 1), _rows(BEF, H), _rows(BEF, H),
                   _rows(BEF, H), _rows(BEF, H)],
        out_shape=[jax.ShapeDtypeStruct((E, 1), jnp.float32),
                   jax.ShapeDtypeStruct((E, H), jnp.float32),
                   jax.ShapeDtypeStruct((E, H), jnp.float32),
                   jax.ShapeDtypeStruct((E, H), jnp.float32),
                   jax.ShapeDtypeStruct((E, H), jnp.float32)],
    )(dsq, bt2,
      edge_mlp_W1, edge_mlp_b1.reshape(1, H), edge_mlp_W2,
      edge_mlp_b2.reshape(1, H), bond_emb,
      edge_cat_W1, edge_cat_b1.reshape(1, H), edge_cat_W2,
      edge_cat_b2.reshape(1, H),
      filt_W1, filt_b1.reshape(L, 1, H), filt_W2, filt_b2.reshape(L, 1, H))

    wfs = (wf0, wf1, wf2)

    h = z
    for l in range(L):
        parts = _sc_msg(hl, wfs[l], row3, col3)
        a0 = parts[:N]
        a1 = parts[N:]
        if l < L - 1:
            h, hl = pl.pallas_call(
                _hupd_body,
                grid=(N // BN,),
                in_specs=[_rows(BN, H), _rows(BN, H), _rows(BN, H),
                          _full((H, H)), _full((1, H)), _full((H, H))],
                out_specs=[_rows(BN, H), _rows(BN, H)],
                out_shape=[jax.ShapeDtypeStruct((N, H), jnp.float32),
                           jax.ShapeDtypeStruct((N, H), jnp.float32)],
            )(h, a0, a1, out_W[l], out_b[l].reshape(1, H), lin_W[l + 1])
        else:
            h = pl.pallas_call(
                _hupd_last_body,
                grid=(N // BN,),
                in_specs=[_rows(BN, H), _rows(BN, H), _rows(BN, H),
                          _full((H, H)), _full((1, H))],
                out_specs=_rows(BN, H),
                out_shape=jax.ShapeDtypeStruct((N, H), jnp.float32),
            )(h, a0, a1, out_W[l], out_b[l].reshape(1, H))

    hh = _sc_pair(h, row3, col3)

    edge_inv = pl.pallas_call(
        _pairmlp_body,
        grid=(E // BEF,),
        in_specs=[_rows(BEF, H), _rows(BEF, H),
                  _full((2 * H, H)), _full((1, H)),
                  _full((H, HALF)), _full((1, HALF)),
                  _full((HALF, 1)), _full((1, 1))],
        out_specs=_rows(BEF, 1),
        out_shape=jax.ShapeDtypeStruct((E, 1), jnp.float32),
    )(hh, ea, g_W1, g_b1.reshape(1, H), g_W2, g_b2.reshape(1, HALF),
      g_W3, g_b3.reshape(1, 1))

    return (edge_inv, bond_index, el)
